# baseline (speedup 1.0000x reference)
import functools

import jax
import jax.numpy as jnp
from jax import lax
from jax.experimental import pallas as pl
from jax.experimental.pallas import tpu as pltpu
from jax.experimental.pallas import tpu_sc as plsc

_MAXP = 512
_EMB = 1024
_SEQ = 4096
_NC = 2
_NS = 16
_NW = _NC * _NS
_RPW = _SEQ // _NW
_CHUNK = 32
_NCHUNK = _RPW // _CHUNK
_LANES = 16


def _sc_lookup(off_hbm, table_hbm, out_hbm, off_v, idx_v, buf0, buf1, sem0, sem1):
    wid = lax.axis_index("s") * _NC + lax.axis_index("c")
    base = wid * _RPW

    pltpu.sync_copy(off_hbm, off_v)
    off = off_v[...]

    iota = lax.iota(jnp.int32, _LANES)
    for j in range(_RPW // _LANES):
        vec = iota + (base + j * _LANES) + off
        clipped = jnp.minimum(jnp.maximum(vec, 0), 2 * _MAXP)
        idx_v[pl.ds(j * _LANES, _LANES)] = clipped

    bufs = (buf0, buf1)
    sems = (sem0, sem1)
    handles = [None] * _NCHUNK

    def start(c):
        handles[c] = pltpu.async_copy(
            table_hbm.at[idx_v.at[pl.ds(c * _CHUNK, _CHUNK)]],
            bufs[c % 2],
            sems[c % 2],
        )

    start(0)
    if _NCHUNK > 1:
        start(1)
    for c in range(_NCHUNK):
        handles[c].wait()
        pltpu.sync_copy(bufs[c % 2], out_hbm.at[pl.ds(base + c * _CHUNK, _CHUNK)])
        if c + 2 < _NCHUNK:
            start(c + 2)


def kernel(seq_len_q, embeddings_table):
    off = jnp.asarray(seq_len_q, jnp.int32) - _SEQ + _MAXP
    off_vec = jnp.full((_LANES,), off, dtype=jnp.int32)

    mesh = plsc.VectorSubcoreMesh(core_axis_name="c", subcore_axis_name="s")
    run = functools.partial(
        pl.kernel,
        mesh=mesh,
        out_type=jax.ShapeDtypeStruct((_SEQ, _EMB), jnp.float32),
        scratch_types=[
            pltpu.VMEM((_LANES,), jnp.int32),
            pltpu.VMEM((_RPW,), jnp.int32),
            pltpu.VMEM((_CHUNK, _EMB), jnp.float32),
            pltpu.VMEM((_CHUNK, _EMB), jnp.float32),
            pltpu.SemaphoreType.DMA,
            pltpu.SemaphoreType.DMA,
        ],
    )(_sc_lookup)
    return run(off_vec, embeddings_table.astype(jnp.float32))


# speedup vs baseline: 5.9077x; 5.9077x over previous
import functools

import jax
import jax.numpy as jnp
from jax import lax
from jax.experimental import pallas as pl
from jax.experimental.pallas import tpu as pltpu
from jax.experimental.pallas import tpu_sc as plsc

_MAXP = 512
_EMB = 1024
_SEQ = 4096
_TOP = 2 * _MAXP
_NC = 2
_NS = 16
_NW = _NC * _NS
_RPW = _SEQ // _NW
_CHUNK = 32
_NCHUNK = _RPW // _CHUNK
_BROWS = 8
_LANES = 16


def _sc_lookup(off_hbm, table_hbm, out_hbm, off_v, idx_v, buf0, buf1, bcast,
               sem0, sem1, semw):
    wid = lax.axis_index("s") * _NC + lax.axis_index("c")
    base = wid * _RPW

    pltpu.sync_copy(off_hbm, off_v)
    s0 = base + off_v[...][0]

    aligned = jnp.bitwise_and(s0, 7) == 0
    whole_in = (s0 >= 0) & (s0 + _RPW - 1 <= _TOP)
    whole_cl = (s0 + _RPW - 1 <= 0) | (s0 >= _TOP)
    fast = whole_in & aligned

    @pl.when(fast)
    def _linear():
        s0a = pl.multiple_of(s0, 8)
        bufs = (buf0, buf1)
        sems = (sem0, sem1)
        handles = [None] * _NCHUNK

        def start(c):
            handles[c] = pltpu.async_copy(
                table_hbm.at[pl.ds(s0a + c * _CHUNK, _CHUNK)],
                bufs[c % 2], sems[c % 2])

        start(0)
        start(1)
        for c in range(_NCHUNK):
            handles[c].wait()
            pltpu.sync_copy(bufs[c % 2],
                            out_hbm.at[pl.ds(base + c * _CHUNK, _CHUNK)])
            if c + 2 < _NCHUNK:
                start(c + 2)

    @pl.when(whole_cl)
    def _broadcast():
        any_low = s0 + _RPW - 1 <= 0
        r_src = pl.multiple_of(jnp.where(any_low, 0, _TOP), 8)
        pltpu.sync_copy(table_hbm.at[pl.ds(r_src, 1)], bcast.at[pl.ds(0, 1)])
        for v in range(_EMB // _LANES):
            row0 = bcast[0, pl.ds(v * _LANES, _LANES)]
            for r in range(1, _BROWS):
                bcast[r, pl.ds(v * _LANES, _LANES)] = row0
        handles = [
            pltpu.async_copy(
                bcast, out_hbm.at[pl.ds(base + k * _BROWS, _BROWS)], semw)
            for k in range(_RPW // _BROWS)
        ]
        for h in handles:
            h.wait()

    @pl.when(jnp.logical_not(fast | whole_cl))
    def _general():
        iota = lax.iota(jnp.int32, _LANES)
        for j in range(_RPW // _LANES):
            vec = iota + j * _LANES + s0
            clipped = jnp.minimum(jnp.maximum(vec, 0), _TOP)
            idx_v[pl.ds(j * _LANES, _LANES)] = clipped

        bufs = (buf0, buf1)
        sems = (sem0, sem1)
        handles = [None] * _NCHUNK

        def start(c):
            handles[c] = pltpu.async_copy(
                table_hbm.at[idx_v.at[pl.ds(c * _CHUNK, _CHUNK)]],
                bufs[c % 2], sems[c % 2])

        start(0)
        start(1)
        for c in range(_NCHUNK):
            handles[c].wait()
            pltpu.sync_copy(bufs[c % 2],
                            out_hbm.at[pl.ds(base + c * _CHUNK, _CHUNK)])
            if c + 2 < _NCHUNK:
                start(c + 2)


def kernel(seq_len_q, embeddings_table):
    off = jnp.asarray(seq_len_q, jnp.int32) - _SEQ + _MAXP
    off_vec = jnp.full((_LANES,), off, dtype=jnp.int32)

    mesh = plsc.VectorSubcoreMesh(core_axis_name="c", subcore_axis_name="s")
    run = functools.partial(
        pl.kernel,
        mesh=mesh,
        out_type=jax.ShapeDtypeStruct((_SEQ, _EMB), jnp.float32),
        scratch_types=[
            pltpu.VMEM((_LANES,), jnp.int32),
            pltpu.VMEM((_RPW,), jnp.int32),
            pltpu.VMEM((_CHUNK, _EMB), jnp.float32),
            pltpu.VMEM((_CHUNK, _EMB), jnp.float32),
            pltpu.VMEM((_BROWS, _EMB), jnp.float32),
            pltpu.SemaphoreType.DMA,
            pltpu.SemaphoreType.DMA,
            pltpu.SemaphoreType.DMA,
        ],
    )(_sc_lookup)
    return run(off_vec, embeddings_table.astype(jnp.float32))
